# baseline (speedup 1.0000x reference)
import jax
import jax.numpy as jnp
from jax import lax
from jax.experimental import pallas as pl
from jax.experimental.pallas import tpu as pltpu
from jax.experimental.pallas import tpu_sc as plsc

H = 16384
F_IN = 16384
B = 64
S = 16
NNZ = 268435

NCORE = 2
NSUBC = 16
NW = NCORE * NSUBC
CH = 128
BAND = F_IN // NW
TOT = NNZ // CH + NW

ROWS_PER_W = H // NW
ROWS_PER_S = H // NSUBC

_mesh = plsc.VectorSubcoreMesh(core_axis_name="c", subcore_axis_name="s")


def _bcast16(i):
    return jnp.full((16,), i, dtype=jnp.int32)


def _spmm_body(x_t, h, ih_ibuf, ih_vbuf, ih_meta, hh_ibuf, hh_vbuf, hh_meta,
               part_out, acc, tband, sb0, sb1, ib0, ib1, vb0, vb1, rb0, rb1,
               mA, isem0, isem1, vsem0, vsem1, ssem0, ssem1):
    c = lax.axis_index("c")
    s = lax.axis_index("s")
    w = c * NSUBC + s

    zero = jnp.zeros((16,), jnp.float32)

    def _zb(i, _):
        for j in range(B // 16):
            sb0[i, pl.ds(j * 16, 16)] = zero
        return 0

    lax.fori_loop(0, CH, _zb, 0, unroll=8)
    for i in range(ROWS_PER_S // CH):
        pltpu.sync_copy(sb0, acc.at[pl.ds(s * ROWS_PER_S + i * CH, CH)])
    plsc.subcore_barrier()

    ii = lax.broadcasted_iota(jnp.int32, (16,), 0)
    ibufs = (ib0, ib1)
    isems = (isem0, isem1)
    vbufs = (vb0, vb1)
    vsems = (vsem0, vsem1)
    rbufs = (rb0, rb1)
    sbufs = (sb0, sb1)
    ssems = (ssem0, ssem1)
    offs = [ii + jnp.int32(j * 16) for j in range(B // 16)]

    def _run_matrix(table, ibuf, vbuf, meta):
        pltpu.sync_copy(meta.at[0, c], mA)
        nchw = jnp.max(jnp.where(ii == s, mA[...], 0))
        pltpu.sync_copy(meta.at[1, c], mA)
        cbw = jnp.max(jnp.where(ii == s, mA[...], 0))

        pltpu.sync_copy(table.at[pl.ds(w * BAND, BAND)], tband)

        for b in range(2):
            @pl.when(nchw > b)
            def _():
                pltpu.async_copy(ibuf.at[cbw + b], ibufs[b], isems[b])
                pltpu.async_copy(vbuf.at[cbw + b], vbufs[b], vsems[b])

        def _scale(ib, vb, sb):
            def _grp(g, _):
                for l in range(16):
                    i = g * 16 + l
                    lv = plsc.load_gather(ib, [_bcast16(1), _bcast16(i)])
                    bv = plsc.load_gather(vb, [_bcast16(i)])
                    a = [plsc.load_gather(tband, [lv, offs[j]])
                         for j in range(B // 16)]
                    for j in range(B // 16):
                        sb[i, pl.ds(j * 16, 16)] = a[j] * bv
                return 0

            lax.fori_loop(0, CH // 16, _grp, 0)

        def _iter(ko, _):
            for b in range(2):
                k = ko * 2 + b
                ib, isem = ibufs[b], isems[b]
                vb, vsem = vbufs[b], vsems[b]
                rb = rbufs[b]
                sb, ssem = sbufs[b], ssems[b]

                @pl.when(k < nchw)
                def _():
                    pltpu.make_async_copy(ibuf.at[cbw + k], ib, isem).wait()
                    pltpu.make_async_copy(vbuf.at[cbw + k], vb, vsem).wait()
                    @pl.when(k >= 2)
                    def _():
                        pltpu.make_async_copy(sb, acc.at[rb], ssem).wait()
                    _scale(ib, vb, sb)
                    for g in range(CH // 16):
                        rb[pl.ds(g * 16, 16)] = ib[0, pl.ds(g * 16, 16)]
                    pltpu.async_copy(sb, acc.at[rb], ssem, add=True)
                    @pl.when(k + 2 < nchw)
                    def _():
                        pltpu.async_copy(ibuf.at[cbw + k + 2], ib, isem)
                        pltpu.async_copy(vbuf.at[cbw + k + 2], vb, vsem)
            return 0

        lax.fori_loop(0, (nchw + 1) // 2, _iter, 0)
        for b in range(2):
            @pl.when(nchw > b)
            def _():
                pltpu.make_async_copy(sbufs[b], acc.at[rbufs[b]],
                                      ssems[b]).wait()

    _run_matrix(x_t, ih_ibuf, ih_vbuf, ih_meta)
    _run_matrix(h, hh_ibuf, hh_vbuf, hh_meta)

    plsc.subcore_barrier()
    for i in range(ROWS_PER_S // CH):
        r = s * ROWS_PER_S + i * CH
        pltpu.sync_copy(acc.at[pl.ds(r, CH)], part_out.at[c, pl.ds(r, CH)])


_params = pltpu.CompilerParams(needs_layout_passes=False,
                               use_tc_tiling_on_sc=False)

_k1 = pl.kernel(
    _spmm_body,
    out_type=jax.ShapeDtypeStruct((NCORE, H, B), jnp.float32),
    mesh=_mesh,
    compiler_params=_params,
    scratch_types=[
        pltpu.VMEM_SHARED((H, B), jnp.float32),
        pltpu.VMEM((BAND, B), jnp.float32),
        pltpu.VMEM((CH, B), jnp.float32),
        pltpu.VMEM((CH, B), jnp.float32),
        pltpu.VMEM((2, CH), jnp.int32),
        pltpu.VMEM((2, CH), jnp.int32),
        pltpu.VMEM((CH,), jnp.float32),
        pltpu.VMEM((CH,), jnp.float32),
        pltpu.VMEM((CH,), jnp.int32),
        pltpu.VMEM((CH,), jnp.int32),
        pltpu.VMEM((16,), jnp.int32),
        pltpu.SemaphoreType.DMA,
        pltpu.SemaphoreType.DMA,
        pltpu.SemaphoreType.DMA,
        pltpu.SemaphoreType.DMA,
        pltpu.SemaphoreType.DMA,
        pltpu.SemaphoreType.DMA,
    ],
)


def _tanh_tc_body(p0, p1, b_ih, b_hh, h_out):
    h_out[...] = jnp.tanh(p0[...] + p1[...] + b_ih[...] + b_hh[...])


_k2 = pl.pallas_call(
    _tanh_tc_body,
    grid=(NW,),
    in_specs=[
        pl.BlockSpec((ROWS_PER_W, B), lambda i: (i, 0)),
        pl.BlockSpec((ROWS_PER_W, B), lambda i: (i, 0)),
        pl.BlockSpec((ROWS_PER_W, 1), lambda i: (i, 0)),
        pl.BlockSpec((ROWS_PER_W, 1), lambda i: (i, 0)),
    ],
    out_specs=pl.BlockSpec((ROWS_PER_W, B), lambda i: (i, 0)),
    out_shape=jax.ShapeDtypeStruct((H, B), jnp.float32),
)


def _prep(rows, cols, vals):
    order = jnp.argsort(cols)
    r = rows[order].astype(jnp.int32)
    c = cols[order].astype(jnp.int32)
    v = vals[order]
    band = c >> 9
    counts = jnp.bincount(band, length=NW)
    nch = (counts + CH - 1) // CH
    choff = jnp.concatenate([jnp.zeros(1, nch.dtype), jnp.cumsum(nch)[:-1]])
    start = jnp.concatenate([jnp.zeros(1, counts.dtype),
                             jnp.cumsum(counts)[:-1]])
    adj = choff * CH - start
    dest = jnp.arange(NNZ, dtype=jnp.int32) + adj[band].astype(jnp.int32)

    rows_a = (jnp.arange(TOT * CH, dtype=jnp.int32) % H).at[dest].set(r)
    lidx_a = jnp.zeros(TOT * CH, jnp.int32).at[dest].set(c & (BAND - 1))
    vals_a = jnp.zeros(TOT * CH, jnp.float32).at[dest].set(v)

    ibuf = jnp.stack([rows_a.reshape(TOT, CH), lidx_a.reshape(TOT, CH)],
                     axis=1)
    vbuf = vals_a.reshape(TOT, CH)
    meta = jnp.stack([nch, choff]).astype(jnp.int32).reshape(2, NCORE, NSUBC)
    return ibuf, vbuf, meta


def kernel(x, ih_vals, bias_ih, hh_vals, bias_hh, ih_rows, ih_cols, hh_rows, hh_cols):
    xp = jnp.transpose(x, (1, 2, 0))
    ih_ibuf, ih_vbuf, ih_meta = _prep(ih_rows, ih_cols, ih_vals)
    hh_ibuf, hh_vbuf, hh_meta = _prep(hh_rows, hh_cols, hh_vals)

    h = jnp.zeros((H, B), jnp.float32)
    outs = []
    for t in range(S):
        part = _k1(xp[t], h, ih_ibuf, ih_vbuf, ih_meta,
                   hh_ibuf, hh_vbuf, hh_meta)
        h = _k2(part[0], part[1], bias_ih, bias_hh)
        outs.append(h)

    out = jnp.transpose(jnp.stack(outs), (2, 0, 1))
    h_final = jnp.transpose(h[None, :, :], (2, 0, 1))
    return (out, h_final)


# speedup vs baseline: 1.2459x; 1.2459x over previous
import jax
import jax.numpy as jnp
from jax import lax
from jax.experimental import pallas as pl
from jax.experimental.pallas import tpu as pltpu
from jax.experimental.pallas import tpu_sc as plsc

H = 16384
F_IN = 16384
B = 64
S = 16
NNZ = 268435

NCORE = 2
NSUBC = 16
NW = NCORE * NSUBC
CH = 128
BAND = F_IN // NW
TOT = NNZ // CH + NW

ROWS_PER_W = H // NW
ROWS_PER_S = H // NSUBC

_mesh = plsc.VectorSubcoreMesh(core_axis_name="c", subcore_axis_name="s")


def _bcast16(i):
    return jnp.full((16,), i, dtype=jnp.int32)


def _spmm_body(x_t, h, ih_ibuf, ih_vbuf, ih_meta, hh_ibuf, hh_vbuf, hh_meta,
               part_out, acc, tband, sb0, sb1, ib0, ib1, vb0, vb1, rb0, rb1,
               mA, isem0, isem1, vsem0, vsem1, ssem0, ssem1):
    c = lax.axis_index("c")
    s = lax.axis_index("s")
    w = c * NSUBC + s

    zero = jnp.zeros((16,), jnp.float32)

    def _zb(i, _):
        for j in range(B // 16):
            sb0[i, pl.ds(j * 16, 16)] = zero
        return 0

    lax.fori_loop(0, CH, _zb, 0, unroll=8)
    for i in range(ROWS_PER_S // CH):
        pltpu.sync_copy(sb0, acc.at[pl.ds(s * ROWS_PER_S + i * CH, CH)])
    plsc.subcore_barrier()

    ii = lax.broadcasted_iota(jnp.int32, (16,), 0)
    ibufs = (ib0, ib1)
    isems = (isem0, isem1)
    vbufs = (vb0, vb1)
    vsems = (vsem0, vsem1)
    rbufs = (rb0, rb1)
    sbufs = (sb0, sb1)
    ssems = (ssem0, ssem1)
    offs = [ii + jnp.int32(j * 16) for j in range(B // 16)]

    def _run_matrix(table, ibuf, vbuf, meta):
        pltpu.sync_copy(meta.at[0, c], mA)
        nchw = jnp.max(jnp.where(ii == s, mA[...], 0))
        pltpu.sync_copy(meta.at[1, c], mA)
        cbw = jnp.max(jnp.where(ii == s, mA[...], 0))

        pltpu.sync_copy(table.at[pl.ds(w * BAND, BAND)], tband)

        for b in range(2):
            @pl.when(nchw > b)
            def _():
                pltpu.async_copy(ibuf.at[cbw + b], ibufs[b], isems[b])
                pltpu.async_copy(vbuf.at[cbw + b], vbufs[b], vsems[b])

        def _scale(ib, vb, sb):
            def _grp(g, _):
                for l in range(16):
                    i = g * 16 + l
                    lv = plsc.load_gather(ib, [_bcast16(1), _bcast16(i)])
                    bv = plsc.load_gather(vb, [_bcast16(i)])
                    a = [plsc.load_gather(tband, [lv, offs[j]])
                         for j in range(B // 16)]
                    for j in range(B // 16):
                        sb[i, pl.ds(j * 16, 16)] = a[j] * bv
                return 0

            lax.fori_loop(0, CH // 16, _grp, 0)

        def _iter(ko, _):
            for b in range(2):
                k = ko * 2 + b
                ib, isem = ibufs[b], isems[b]
                vb, vsem = vbufs[b], vsems[b]
                rb = rbufs[b]
                sb, ssem = sbufs[b], ssems[b]

                @pl.when(k < nchw)
                def _():
                    pltpu.make_async_copy(ibuf.at[cbw + k], ib, isem).wait()
                    pltpu.make_async_copy(vbuf.at[cbw + k], vb, vsem).wait()
                    @pl.when(k >= 2)
                    def _():
                        pltpu.make_async_copy(sb, acc.at[rb], ssem).wait()
                    _scale(ib, vb, sb)
                    for g in range(CH // 16):
                        rb[pl.ds(g * 16, 16)] = ib[0, pl.ds(g * 16, 16)]
                    pltpu.async_copy(sb, acc.at[rb], ssem, add=True)
                    @pl.when(k + 2 < nchw)
                    def _():
                        pltpu.async_copy(ibuf.at[cbw + k + 2], ib, isem)
                        pltpu.async_copy(vbuf.at[cbw + k + 2], vb, vsem)
            return 0

        lax.fori_loop(0, (nchw + 1) // 2, _iter, 0)
        for b in range(2):
            @pl.when(nchw > b)
            def _():
                pltpu.make_async_copy(sbufs[b], acc.at[rbufs[b]],
                                      ssems[b]).wait()

    _run_matrix(x_t, ih_ibuf, ih_vbuf, ih_meta)
    _run_matrix(h, hh_ibuf, hh_vbuf, hh_meta)

    plsc.subcore_barrier()
    for i in range(ROWS_PER_S // CH):
        r = s * ROWS_PER_S + i * CH
        pltpu.sync_copy(acc.at[pl.ds(r, CH)], part_out.at[c, pl.ds(r, CH)])


_params = pltpu.CompilerParams(needs_layout_passes=False,
                               use_tc_tiling_on_sc=False)

_k1 = pl.kernel(
    _spmm_body,
    out_type=jax.ShapeDtypeStruct((NCORE, H, B), jnp.float32),
    mesh=_mesh,
    compiler_params=_params,
    scratch_types=[
        pltpu.VMEM_SHARED((H, B), jnp.float32),
        pltpu.VMEM((BAND, B), jnp.float32),
        pltpu.VMEM((CH, B), jnp.float32),
        pltpu.VMEM((CH, B), jnp.float32),
        pltpu.VMEM((2, CH), jnp.int32),
        pltpu.VMEM((2, CH), jnp.int32),
        pltpu.VMEM((CH,), jnp.float32),
        pltpu.VMEM((CH,), jnp.float32),
        pltpu.VMEM((CH,), jnp.int32),
        pltpu.VMEM((CH,), jnp.int32),
        pltpu.VMEM((16,), jnp.int32),
        pltpu.SemaphoreType.DMA,
        pltpu.SemaphoreType.DMA,
        pltpu.SemaphoreType.DMA,
        pltpu.SemaphoreType.DMA,
        pltpu.SemaphoreType.DMA,
        pltpu.SemaphoreType.DMA,
    ],
)


def _tanh_tc_body(p0, p1, b_ih, b_hh, h_out):
    h_out[...] = jnp.tanh(p0[...] + p1[...] + b_ih[...] + b_hh[...])


_k2 = pl.pallas_call(
    _tanh_tc_body,
    grid=(NW,),
    in_specs=[
        pl.BlockSpec((ROWS_PER_W, B), lambda i: (i, 0)),
        pl.BlockSpec((ROWS_PER_W, B), lambda i: (i, 0)),
        pl.BlockSpec((ROWS_PER_W, 1), lambda i: (i, 0)),
        pl.BlockSpec((ROWS_PER_W, 1), lambda i: (i, 0)),
    ],
    out_specs=pl.BlockSpec((ROWS_PER_W, B), lambda i: (i, 0)),
    out_shape=jax.ShapeDtypeStruct((H, B), jnp.float32),
)


def _prep(rows, cols, vals):
    r = rows.astype(jnp.int32)
    c = cols.astype(jnp.int32)
    v = vals
    band = c >> 9
    oh = (band[:, None] == jnp.arange(NW, dtype=band.dtype)[None, :])
    cum = jnp.cumsum(oh.astype(jnp.int32), axis=0)
    counts = cum[-1]
    pos = jnp.take_along_axis(cum, band[:, None], axis=1)[:, 0] - 1
    nch = (counts + CH - 1) // CH
    choff = jnp.concatenate([jnp.zeros(1, nch.dtype), jnp.cumsum(nch)[:-1]])
    dest = choff[band].astype(jnp.int32) * CH + pos

    rows_a = (jnp.arange(TOT * CH, dtype=jnp.int32) % H).at[dest].set(r)
    lidx_a = jnp.zeros(TOT * CH, jnp.int32).at[dest].set(c & (BAND - 1))
    vals_a = jnp.zeros(TOT * CH, jnp.float32).at[dest].set(v)

    ibuf = jnp.stack([rows_a.reshape(TOT, CH), lidx_a.reshape(TOT, CH)],
                     axis=1)
    vbuf = vals_a.reshape(TOT, CH)
    meta = jnp.stack([nch, choff]).astype(jnp.int32).reshape(2, NCORE, NSUBC)
    return ibuf, vbuf, meta


def kernel(x, ih_vals, bias_ih, hh_vals, bias_hh, ih_rows, ih_cols, hh_rows, hh_cols):
    xp = jnp.transpose(x, (1, 2, 0))
    ih_ibuf, ih_vbuf, ih_meta = _prep(ih_rows, ih_cols, ih_vals)
    hh_ibuf, hh_vbuf, hh_meta = _prep(hh_rows, hh_cols, hh_vals)

    h = jnp.zeros((H, B), jnp.float32)
    outs = []
    for t in range(S):
        part = _k1(xp[t], h, ih_ibuf, ih_vbuf, ih_meta,
                   hh_ibuf, hh_vbuf, hh_meta)
        h = _k2(part[0], part[1], bias_ih, bias_hh)
        outs.append(h)

    out = jnp.transpose(jnp.stack(outs), (2, 0, 1))
    h_final = jnp.transpose(h[None, :, :], (2, 0, 1))
    return (out, h_final)


# speedup vs baseline: 5.1336x; 4.1204x over previous
import jax
import jax.numpy as jnp
from jax import lax
from jax.experimental import pallas as pl
from jax.experimental.pallas import tpu as pltpu
from jax.experimental.pallas import tpu_sc as plsc

H = 16384
F_IN = 16384
B = 64
S = 16
NNZ = 268435

NCORE = 2
NSUBC = 16
NW = NCORE * NSUBC
CH = 128
NCH = -(-NNZ // (NW * CH))
NNZ_PAD = NW * NCH * CH

ROWS_PER_W = H // NW
ROWS_PER_S = H // NSUBC

_mesh = plsc.VectorSubcoreMesh(core_axis_name="c", subcore_axis_name="s")


def _bcast16(i):
    return jnp.full((16,), i, dtype=jnp.int32)


def _spmm_body(x_t, h, ih_cols, ih_rows, ih_vals, hh_cols, hh_rows, hh_vals,
               part_out, acc, gb0, gb1, sbuf, cA, rA, vA, gsem0, gsem1):
    c = lax.axis_index("c")
    s = lax.axis_index("s")
    w = c * NSUBC + s

    zero = jnp.zeros((16,), jnp.float32)

    def _zb(i, _):
        for j in range(B // 16):
            gb0[i, pl.ds(j * 16, 16)] = zero
        return 0

    lax.fori_loop(0, CH, _zb, 0, unroll=8)
    for i in range(ROWS_PER_S // CH):
        pltpu.sync_copy(gb0, acc.at[pl.ds(s * ROWS_PER_S + i * CH, CH)])
    plsc.subcore_barrier()

    gbufs = (gb0, gb1)
    gsems = (gsem0, gsem1)

    def _run_matrix(table, cols3, rows3, vals3):
        pltpu.sync_copy(cols3.at[w], cA)
        pltpu.sync_copy(rows3.at[w], rA)
        pltpu.sync_copy(vals3.at[w], vA)

        def _scale(gb, k):
            def _grp(g, _):
                for l in range(16):
                    i = g * 16 + l
                    bv = plsc.load_gather(vA, [_bcast16(k), _bcast16(i)])
                    a = [gb[i, pl.ds(j * 16, 16)] for j in range(B // 16)]
                    for j in range(B // 16):
                        sbuf[i, pl.ds(j * 16, 16)] = a[j] * bv
                return 0

            lax.fori_loop(0, CH // 16, _grp, 0)

        pltpu.async_copy(table.at[cA.at[0]], gb0, gsem0)

        def _iter(ko, _):
            for b in range(2):
                k = ko * 2 + b
                gb, gsem = gbufs[b], gsems[b]
                ob, osem = gbufs[1 - b], gsems[1 - b]
                pltpu.make_async_copy(table.at[cA.at[k]], gb, gsem).wait()
                if b == 0:
                    pltpu.async_copy(table.at[cA.at[k + 1]], ob, osem)
                else:
                    @pl.when(ko < NCH // 2 - 1)
                    def _():
                        pltpu.async_copy(table.at[cA.at[k + 1]], ob, osem)
                _scale(gb, k)
                pltpu.sync_copy(sbuf, acc.at[rA.at[k]], add=True)
            return 0

        lax.fori_loop(0, NCH // 2, _iter, 0)

    _run_matrix(x_t, ih_cols, ih_rows, ih_vals)
    _run_matrix(h, hh_cols, hh_rows, hh_vals)

    plsc.subcore_barrier()
    for i in range(ROWS_PER_S // CH):
        r = s * ROWS_PER_S + i * CH
        pltpu.sync_copy(acc.at[pl.ds(r, CH)], part_out.at[c, pl.ds(r, CH)])


_params = pltpu.CompilerParams(needs_layout_passes=False,
                               use_tc_tiling_on_sc=False)

_k1 = pl.kernel(
    _spmm_body,
    out_type=jax.ShapeDtypeStruct((NCORE, H, B), jnp.float32),
    mesh=_mesh,
    compiler_params=_params,
    scratch_types=[
        pltpu.VMEM_SHARED((H, B), jnp.float32),
        pltpu.VMEM((CH, B), jnp.float32),
        pltpu.VMEM((CH, B), jnp.float32),
        pltpu.VMEM((CH, B), jnp.float32),
        pltpu.VMEM((NCH, CH), jnp.int32),
        pltpu.VMEM((NCH, CH), jnp.int32),
        pltpu.VMEM((NCH, CH), jnp.float32),
        pltpu.SemaphoreType.DMA,
        pltpu.SemaphoreType.DMA,
    ],
)


def _tanh_tc_body(p0, p1, b_ih, b_hh, h_out):
    h_out[...] = jnp.tanh(p0[...] + p1[...] + b_ih[...] + b_hh[...])


_k2 = pl.pallas_call(
    _tanh_tc_body,
    grid=(NW,),
    in_specs=[
        pl.BlockSpec((ROWS_PER_W, B), lambda i: (i, 0)),
        pl.BlockSpec((ROWS_PER_W, B), lambda i: (i, 0)),
        pl.BlockSpec((ROWS_PER_W, 1), lambda i: (i, 0)),
        pl.BlockSpec((ROWS_PER_W, 1), lambda i: (i, 0)),
    ],
    out_specs=pl.BlockSpec((ROWS_PER_W, B), lambda i: (i, 0)),
    out_shape=jax.ShapeDtypeStruct((H, B), jnp.float32),
)


def _prep(rows, cols, vals):
    pad = NNZ_PAD - NNZ
    rows = jnp.pad(rows, (0, pad)).reshape(NW, NCH, CH)
    cols = jnp.pad(cols, (0, pad)).reshape(NW, NCH, CH)
    vals = jnp.pad(vals, (0, pad)).reshape(NW, NCH, CH)
    return rows, cols, vals


def kernel(x, ih_vals, bias_ih, hh_vals, bias_hh, ih_rows, ih_cols, hh_rows, hh_cols):
    xp = jnp.transpose(x, (1, 2, 0))
    ihr, ihc, ihv = _prep(ih_rows, ih_cols, ih_vals)
    hhr, hhc, hhv = _prep(hh_rows, hh_cols, hh_vals)

    h = jnp.zeros((H, B), jnp.float32)
    outs = []
    for t in range(S):
        part = _k1(xp[t], h, ihc, ihr, ihv, hhc, hhr, hhv)
        h = _k2(part[0], part[1], bias_ih, bias_hh)
        outs.append(h)

    out = jnp.transpose(jnp.stack(outs), (2, 0, 1))
    h_final = jnp.transpose(h[None, :, :], (2, 0, 1))
    return (out, h_final)
